# baseline (speedup 1.0000x reference)
import numpy as np
import jax
import jax.numpy as jnp
from jax.experimental import pallas as pl

NUM_ITEMS = 1000000
BATCH = 16384
NUM_NEG = 4 * BATCH
TOP = 16384
BIG = np.int32(2**30)


def _rotl(x, d):
    return ((x << np.uint32(d)) | (x >> np.uint32(32 - d))).astype(np.uint32)


def _threefry2x32(k0, k1, x0, x1):
    x0 = x0.astype(np.uint32).copy()
    x1 = x1.astype(np.uint32).copy()
    ks = [np.uint32(k0), np.uint32(k1),
          np.uint32(np.uint32(k0) ^ np.uint32(k1) ^ np.uint32(0x1BD11BDA))]
    rotations = [[13, 15, 26, 6], [17, 29, 16, 24]]
    x0 = (x0 + ks[0]).astype(np.uint32)
    x1 = (x1 + ks[1]).astype(np.uint32)
    for i in range(5):
        for r in rotations[i % 2]:
            x0 = (x0 + x1).astype(np.uint32)
            x1 = _rotl(x1, r)
            x1 = (x1 ^ x0).astype(np.uint32)
        x0 = (x0 + ks[(i + 1) % 3]).astype(np.uint32)
        x1 = (x1 + ks[(i + 2) % 3] + np.uint32(i + 1)).astype(np.uint32)
    return x0, x1


def _np_random_bits(keypair, n):
    hi = np.zeros(n, dtype=np.uint32)
    lo = np.arange(n, dtype=np.uint32)
    o0, o1 = _threefry2x32(keypair[0], keypair[1], hi, lo)
    return o0 ^ o1


def _np_split(keypair):
    hi = np.zeros(2, dtype=np.uint32)
    lo = np.arange(2, dtype=np.uint32)
    o0, o1 = _threefry2x32(keypair[0], keypair[1], hi, lo)
    return (o0[0], o1[0]), (o0[1], o1[1])


def _precompute():
    kp = (np.uint32(0), np.uint32(42))
    kp, sub1 = _np_split(kp)
    bits1 = _np_random_bits(sub1, NUM_ITEMS)
    kp, sub2 = _np_split(kp)
    bits2 = _np_random_bits(sub2, NUM_ITEMS)
    a1 = np.argsort(bits1, kind="stable").astype(np.int32)
    a2 = np.argsort(bits2, kind="stable").astype(np.int32)
    r1g = np.empty(NUM_ITEMS, np.int32)
    r1g[a1] = np.arange(NUM_ITEMS, dtype=np.int32)
    r2g = np.empty(NUM_ITEMS, np.int32)
    r2g[a2] = np.arange(NUM_ITEMS, dtype=np.int32)
    js = np.arange(NUM_ITEMS - TOP, NUM_ITEMS, dtype=np.int32)
    o1_ = np.argsort(r1g[js], kind="stable")
    o2_ = np.argsort(r2g[js], kind="stable")
    return (a1, a2, r1g[js][o1_], js[o1_], r2g[js][o2_], js[o2_])


_A1, _A2, _L1R, _L1J, _L2R, _L2J = _precompute()


def _searchsorted_right(e, q):
    lo = jnp.zeros_like(q)
    hi = jnp.full_like(q, TOP)
    for _ in range(15):
        mid = (lo + hi) >> 1
        v = e[mid]
        go = v <= q
        lo = jnp.where(go, mid + 1, lo)
        hi = jnp.where(go, hi, mid)
    return lo


def kernel(item, rating):
    a1 = jnp.asarray(_A1)
    a2 = jnp.asarray(_A2)
    l1r = jnp.asarray(_L1R)
    l1j = jnp.asarray(_L1J)
    l2r = jnp.asarray(_L2R)
    l2j = jnp.asarray(_L2J)

    u = jnp.unique(item, size=BATCH, fill_value=BIG)
    du = jnp.sum(u < BIG).astype(jnp.int32)
    n = NUM_ITEMS - du
    ar = jnp.arange(TOP, dtype=jnp.int32)
    eu = jnp.where(u < BIG, u - ar, BIG)

    def build_e(lr, lj):
        sel = lj >= n
        rank = jnp.cumsum(sel.astype(jnp.int32)) - 1
        pos = jnp.where(sel, rank, TOP)
        e = jnp.full((TOP + 1,), BIG, jnp.int32).at[pos].set(lr - rank)
        return e[:TOP]

    e1 = build_e(l1r, l1j)
    e2 = build_e(l2r, l2j)

    k = jnp.arange(NUM_NEG, dtype=jnp.int32)
    i = a2[k + _searchsorted_right(e2, k)]
    p = a1[i + _searchsorted_right(e1, i)]
    neg = p + _searchsorted_right(eu, p)

    out_item = jnp.concatenate([item, neg.astype(jnp.int32)])
    out_rating = jnp.concatenate([rating, jnp.zeros((NUM_NEG,), rating.dtype)])
    return out_item, out_rating


# speedup vs baseline: 118.9143x; 118.9143x over previous
import functools

import numpy as np
import jax
import jax.numpy as jnp
from jax import lax
from jax.experimental import pallas as pl
from jax.experimental.pallas import tpu as pltpu
from jax.experimental.pallas import tpu_sc as plsc

NUM_ITEMS = 1000000
BATCH = 16384
NUM_NEG = 4 * BATCH
TOP = 16384
BIG = 2 ** 30
NW = 32
WPT = 1024
VRANGE = 32768
NWORDS = 32768
REAL_WORDS = NUM_ITEMS // 32
QPT = NUM_NEG // NW


def _rotl(x, d):
    return ((x << np.uint32(d)) | (x >> np.uint32(32 - d))).astype(np.uint32)


def _threefry2x32(k0, k1, x0, x1):
    x0 = x0.astype(np.uint32).copy()
    x1 = x1.astype(np.uint32).copy()
    ks = [np.uint32(k0), np.uint32(k1),
          np.uint32(np.uint32(k0) ^ np.uint32(k1) ^ np.uint32(0x1BD11BDA))]
    rotations = [[13, 15, 26, 6], [17, 29, 16, 24]]
    x0 = (x0 + ks[0]).astype(np.uint32)
    x1 = (x1 + ks[1]).astype(np.uint32)
    for i in range(5):
        for r in rotations[i % 2]:
            x0 = (x0 + x1).astype(np.uint32)
            x1 = _rotl(x1, r)
            x1 = (x1 ^ x0).astype(np.uint32)
        x0 = (x0 + ks[(i + 1) % 3]).astype(np.uint32)
        x1 = (x1 + ks[(i + 2) % 3] + np.uint32(i + 1)).astype(np.uint32)
    return x0, x1


def _np_random_bits(keypair, n):
    hi = np.zeros(n, dtype=np.uint32)
    lo = np.arange(n, dtype=np.uint32)
    o0, o1 = _threefry2x32(keypair[0], keypair[1], hi, lo)
    return o0 ^ o1


def _np_split(keypair):
    hi = np.zeros(2, dtype=np.uint32)
    lo = np.arange(2, dtype=np.uint32)
    o0, o1 = _threefry2x32(keypair[0], keypair[1], hi, lo)
    return (o0[0], o1[0]), (o0[1], o1[1])


def _precompute():
    kp = (np.uint32(0), np.uint32(42))
    kp, sub1 = _np_split(kp)
    bits1 = _np_random_bits(sub1, NUM_ITEMS)
    kp, sub2 = _np_split(kp)
    bits2 = _np_random_bits(sub2, NUM_ITEMS)
    a1 = np.argsort(bits1, kind="stable").astype(np.int32)
    a2 = np.argsort(bits2, kind="stable").astype(np.int32)
    r1g = np.empty(NUM_ITEMS, np.int32)
    r1g[a1] = np.arange(NUM_ITEMS, dtype=np.int32)
    r2g = np.empty(NUM_ITEMS, np.int32)
    r2g[a2] = np.arange(NUM_ITEMS, dtype=np.int32)
    js = np.arange(NUM_ITEMS - TOP, NUM_ITEMS, dtype=np.int32)
    o1_ = np.argsort(r1g[js], kind="stable")
    o2_ = np.argsort(r2g[js], kind="stable")
    return (a1, a2, r1g[js][o1_].copy(), js[o1_].copy(),
            r2g[js][o2_].copy(), js[o2_].copy())


_A1, _A2, _L1R, _L1J, _L2R, _L2J = _precompute()

_mesh = plsc.VectorSubcoreMesh(core_axis_name="c", subcore_axis_name="s")
_i32 = jnp.int32


def _wid():
    return lax.axis_index("s") * 2 + lax.axis_index("c")


def _iota16():
    return lax.iota(_i32, 16)


@functools.partial(
    pl.kernel,
    out_type=(
        jax.ShapeDtypeStruct((NWORDS,), _i32),
        jax.ShapeDtypeStruct((NWORDS,), _i32),
        jax.ShapeDtypeStruct((NW, 16), _i32),
    ),
    mesh=_mesh,
    compiler_params=pltpu.CompilerParams(needs_layout_passes=False),
    scratch_types=[
        pltpu.VMEM((BATCH,), _i32),
        pltpu.VMEM((VRANGE,), _i32),
        pltpu.VMEM((WPT,), _i32),
        pltpu.VMEM((WPT,), _i32),
        pltpu.VMEM((WPT,), _i32),
        pltpu.VMEM((16,), _i32),
    ],
)
def _k1(item_hbm, bits_hbm, pcl_hbm, counts_hbm,
        item_v, table_v, words_v, cnts_v, pfx_v, stage_v):
    wid = _wid()
    lo = wid * VRANGE
    iota = _iota16()
    zeros16 = jnp.zeros((16,), _i32)
    ones16 = jnp.ones((16,), _i32)

    pltpu.sync_copy(item_hbm, item_v)

    def zero_body(i, _):
        table_v[pl.ds(i * 16, 16)] = zeros16
        return 0
    lax.fori_loop(0, VRANGE // 16, zero_body, 0)

    def scat_body(i, _):
        v = item_v[pl.ds(i * 16, 16)]
        rel = v - lo
        msk = (rel >= 0) & (rel < VRANGE)
        relc = jnp.clip(rel, 0, VRANGE - 1)
        plsc.store_scatter(table_v, [relc], ones16, mask=msk)
        return 0
    lax.fori_loop(0, BATCH // 16, scat_body, 0)

    def pack_body(g, du_acc):
        widx = g * 16 + iota
        base = widx * 32
        wv = zeros16
        cnt = zeros16
        for l in range(32):
            t = plsc.load_gather(table_v, [base + l])
            nz = t != 0
            bit = np.int32(np.uint32(1 << l))
            wv = wv | jnp.where(nz, jnp.full((16,), bit, _i32), zeros16)
            cnt = cnt + nz.astype(_i32)
        wglob = widx + wid * WPT
        fake = wglob >= REAL_WORDS
        wv = jnp.where(fake, jnp.full((16,), np.int32(-1), _i32), wv)
        du_acc = du_acc + jnp.where(fake, zeros16, cnt)
        cnt = jnp.where(fake, jnp.full((16,), np.int32(32), _i32), cnt)
        words_v[pl.ds(g * 16, 16)] = wv
        cnts_v[pl.ds(g * 16, 16)] = cnt
        return du_acc
    du_acc = lax.fori_loop(0, WPT // 16, pack_body, zeros16)
    du = jnp.sum(du_acc)

    def pfx_body(g, acc):
        c = cnts_v[pl.ds(g * 16, 16)]
        s = plsc.cumsum(c)
        pfx_v[pl.ds(g * 16, 16)] = (s - c) + acc
        return acc + jnp.sum(c)
    total = lax.fori_loop(0, WPT // 16, pfx_body, jnp.int32(0))

    row = jnp.where(iota == 0, du, jnp.where(iota == 1, total, 0)).astype(_i32)
    stage_v[...] = row
    pltpu.sync_copy(stage_v, counts_hbm.at[wid])
    pltpu.sync_copy(words_v, bits_hbm.at[pl.ds(wid * WPT, WPT)])
    pltpu.sync_copy(pfx_v, pcl_hbm.at[pl.ds(wid * WPT, WPT)])


@functools.partial(
    pl.kernel,
    out_type=(
        jax.ShapeDtypeStruct((TOP,), _i32),
        jax.ShapeDtypeStruct((TOP,), _i32),
        jax.ShapeDtypeStruct((NW,), _i32),
    ),
    mesh=_mesh,
    compiler_params=pltpu.CompilerParams(needs_layout_passes=False),
    scratch_types=[
        pltpu.VMEM((NW, 16), _i32),
        pltpu.VMEM((TOP,), _i32),
        pltpu.VMEM((TOP,), _i32),
        pltpu.VMEM((TOP,), _i32),
        pltpu.VMEM((32,), _i32),
    ],
)
def _k2(counts_hbm, l1r_hbm, l1j_hbm, l2r_hbm, l2j_hbm,
        e1_hbm, e2_hbm, pcoff_hbm,
        counts_v, lr_v, lj_v, e_v, stage_v):
    wid = _wid()
    iota = _iota16()
    big16 = jnp.full((16,), np.int32(BIG), _i32)

    pltpu.sync_copy(counts_hbm, counts_v)

    du = jnp.int32(0)
    for r in range(NW):
        v = counts_v[r]
        du = du + jnp.sum(jnp.where(iota == 0, v, 0))
    n = jnp.int32(NUM_ITEMS) - du

    def build_e(r_hbm, j_hbm, out_hbm):
        pltpu.sync_copy(r_hbm, lr_v)
        pltpu.sync_copy(j_hbm, lj_v)

        def fill_body(i, _):
            e_v[pl.ds(i * 16, 16)] = big16
            return 0
        lax.fori_loop(0, TOP // 16, fill_body, 0)

        def comp_body(i, off_v):
            vj = lj_v[pl.ds(i * 16, 16)]
            vr = lr_v[pl.ds(i * 16, 16)]
            m = vj >= n
            ci = plsc.cumsum(m.astype(_i32))
            idx = off_v + ci - 1
            idxc = jnp.maximum(idx, 0)
            plsc.store_scatter(e_v, [idxc], vr - idxc, mask=m)
            return off_v + plsc.all_reduce_population_count(m)
        lax.fori_loop(0, TOP // 16, comp_body, jnp.zeros((16,), _i32))
        pltpu.sync_copy(e_v, out_hbm)

    @pl.when(wid == 0)
    def _():
        build_e(l1r_hbm, l1j_hbm, e1_hbm)

    @pl.when(wid == 1)
    def _():
        build_e(l2r_hbm, l2j_hbm, e2_hbm)

    @pl.when(wid == 2)
    def _():
        vec0 = jnp.zeros((16,), _i32)
        vec1 = jnp.zeros((16,), _i32)
        acc = jnp.int32(0)
        for r in range(NW):
            v = counts_v[r]
            t = jnp.sum(jnp.where(iota == 1, v, 0))
            if r < 16:
                vec0 = jnp.where(iota == r, acc, vec0)
            else:
                vec1 = jnp.where(iota == (r - 16), acc, vec1)
            acc = acc + t
        stage_v[pl.ds(0, 16)] = vec0
        stage_v[pl.ds(16, 16)] = vec1
        pltpu.sync_copy(stage_v, pcoff_hbm)


def _bsearch(e_ref, q):
    lo = jnp.zeros((16,), _i32)
    hi = jnp.full((16,), np.int32(TOP), _i32)
    for _ in range(15):
        mid = (lo + hi) >> 1
        v = plsc.load_gather(e_ref, [mid])
        go = v <= q
        lo = jnp.where(go, mid + 1, lo)
        hi = jnp.where(go, hi, mid)
    return lo


@functools.partial(
    pl.kernel,
    out_type=jax.ShapeDtypeStruct((NUM_NEG,), _i32),
    mesh=_mesh,
    compiler_params=pltpu.CompilerParams(needs_layout_passes=False),
    scratch_types=[
        pltpu.VMEM((TOP,), _i32),
        pltpu.VMEM((TOP,), _i32),
        pltpu.VMEM((NWORDS,), _i32),
        pltpu.VMEM((NWORDS,), _i32),
        pltpu.VMEM((NW,), _i32),
        pltpu.VMEM((QPT,), _i32),
        pltpu.VMEM((16, 128), _i32),
        pltpu.VMEM((QPT,), _i32),
        pltpu.VMEM((QPT,), _i32),
        pltpu.VMEM((QPT,), _i32),
        pltpu.SemaphoreType.DMA,
    ],
)
def _k3(e1_hbm, e2_hbm, bits_hbm, pcl_hbm, pcoff_hbm, a1_hbm, a2_hbm,
        neg_hbm,
        e1_v, e2_v, bits_v, pcg_v, pcoff_v, idxf_v, idx2_v, g2_v, g1_v,
        out_v, sem):
    wid = _wid()
    iota = _iota16()
    kbase = wid * QPT

    pltpu.sync_copy(e1_hbm, e1_v)
    pltpu.sync_copy(e2_hbm, e2_v)
    pltpu.sync_copy(bits_hbm, bits_v)
    pltpu.sync_copy(pcl_hbm, pcg_v)
    pltpu.sync_copy(pcoff_hbm, pcoff_v)

    def glob_body(i, _):
        off = plsc.load_gather(pcoff_v, [jnp.full((16,), lax.div(i, 64), _i32)])
        x = pcg_v[pl.ds(i * 16, 16)]
        pcg_v[pl.ds(i * 16, 16)] = x + off
        return 0
    lax.fori_loop(0, NWORDS // 16, glob_body, 0)

    def q1_body(g, _):
        k = kbase + g * 16 + iota
        off2 = _bsearch(e2_v, k)
        idxf_v[pl.ds(g * 16, 16)] = k + off2
        return 0
    lax.fori_loop(0, QPT // 16, q1_body, 0)

    def stage_idx():
        for r in range(16):
            for c in range(8):
                idx2_v[r, pl.ds(c * 16, 16)] = idxf_v[pl.ds(r * 128 + c * 16, 16)]

    def fire_gather(src_hbm, dst_v):
        cps = [pltpu.async_copy(src_hbm.at[idx2_v.at[r]],
                                dst_v.at[pl.ds(r * 128, 128)], sem)
               for r in range(16)]
        for cp in cps:
            cp.wait()

    stage_idx()
    fire_gather(a2_hbm, g1_v)

    def q2_body(g, _):
        i_v = g1_v[pl.ds(g * 16, 16)]
        off1 = _bsearch(e1_v, i_v)
        idxf_v[pl.ds(g * 16, 16)] = i_v + off1
        return 0
    lax.fori_loop(0, QPT // 16, q2_body, 0)

    stage_idx()
    fire_gather(a1_hbm, g2_v)

    srl = lax.shift_right_logical

    def q3_body(g, _):
        p = g2_v[pl.ds(g * 16, 16)]
        j = jnp.zeros((16,), _i32)
        for s in (16384, 8192, 4096, 2048, 1024, 512, 256, 128,
                  64, 32, 16, 8, 4, 2, 1):
            cand = j + s
            pcv = plsc.load_gather(pcg_v, [cand])
            z = (cand << 5) - pcv
            j = jnp.where(z <= p, cand, j)
        pcj = plsc.load_gather(pcg_v, [j])
        t = p - ((j << 5) - pcj)
        w = plsc.load_gather(bits_v, [j])
        winv = ~w
        b2 = winv - (srl(winv, 1) & 0x55555555)
        b4 = (b2 & 0x33333333) + (srl(b2, 2) & 0x33333333)
        b8 = (b4 + srl(b4, 4)) & 0x0F0F0F0F
        base = jnp.zeros((16,), _i32)
        c16 = (b8 & 0xFF) + (srl(b8, 8) & 0xFF)
        go = (t >= c16).astype(_i32)
        t = t - go * c16
        base = base + go * 16
        c8 = srl(b8, base) & 0xFF
        go = (t >= c8).astype(_i32)
        t = t - go * c8
        base = base + go * 8
        c4 = srl(b4, base) & 0x0F
        go = (t >= c4).astype(_i32)
        t = t - go * c4
        base = base + go * 4
        c2 = srl(b2, base) & 0x03
        go = (t >= c2).astype(_i32)
        t = t - go * c2
        base = base + go * 2
        c1 = srl(winv, base) & 1
        go = (t >= c1).astype(_i32)
        base = base + go
        out_v[pl.ds(g * 16, 16)] = (j << 5) + base
        return 0
    lax.fori_loop(0, QPT // 16, q3_body, 0)

    pltpu.sync_copy(out_v, neg_hbm.at[pl.ds(wid * QPT, QPT)])


def kernel(item, rating):
    a1 = jnp.asarray(_A1)
    a2 = jnp.asarray(_A2)
    l1r = jnp.asarray(_L1R)
    l1j = jnp.asarray(_L1J)
    l2r = jnp.asarray(_L2R)
    l2j = jnp.asarray(_L2J)

    bits, pcl, counts = _k1(item)
    e1, e2, pcoff = _k2(counts, l1r, l1j, l2r, l2j)
    neg = _k3(e1, e2, bits, pcl, pcoff, a1, a2)

    out_item = jnp.concatenate([item, neg])
    out_rating = jnp.concatenate([rating, jnp.zeros((NUM_NEG,), rating.dtype)])
    return out_item, out_rating


# speedup vs baseline: 181.6480x; 1.5276x over previous
import functools

import numpy as np
import jax
import jax.numpy as jnp
from jax import lax
from jax.experimental import pallas as pl
from jax.experimental.pallas import tpu as pltpu
from jax.experimental.pallas import tpu_sc as plsc

NUM_ITEMS = 1000000
BATCH = 16384
NUM_NEG = 4 * BATCH
TOP = 16384
BIG = 2 ** 30
NW = 32
WPT = 1024
VRANGE = 32768
NWORDS = 32768
REAL_WORDS = NUM_ITEMS // 32
QPT = NUM_NEG // NW


def _rotl(x, d):
    return ((x << np.uint32(d)) | (x >> np.uint32(32 - d))).astype(np.uint32)


def _threefry2x32(k0, k1, x0, x1):
    x0 = x0.astype(np.uint32).copy()
    x1 = x1.astype(np.uint32).copy()
    ks = [np.uint32(k0), np.uint32(k1),
          np.uint32(np.uint32(k0) ^ np.uint32(k1) ^ np.uint32(0x1BD11BDA))]
    rotations = [[13, 15, 26, 6], [17, 29, 16, 24]]
    x0 = (x0 + ks[0]).astype(np.uint32)
    x1 = (x1 + ks[1]).astype(np.uint32)
    for i in range(5):
        for r in rotations[i % 2]:
            x0 = (x0 + x1).astype(np.uint32)
            x1 = _rotl(x1, r)
            x1 = (x1 ^ x0).astype(np.uint32)
        x0 = (x0 + ks[(i + 1) % 3]).astype(np.uint32)
        x1 = (x1 + ks[(i + 2) % 3] + np.uint32(i + 1)).astype(np.uint32)
    return x0, x1


def _np_random_bits(keypair, n):
    hi = np.zeros(n, dtype=np.uint32)
    lo = np.arange(n, dtype=np.uint32)
    o0, o1 = _threefry2x32(keypair[0], keypair[1], hi, lo)
    return o0 ^ o1


def _np_split(keypair):
    hi = np.zeros(2, dtype=np.uint32)
    lo = np.arange(2, dtype=np.uint32)
    o0, o1 = _threefry2x32(keypair[0], keypair[1], hi, lo)
    return (o0[0], o1[0]), (o0[1], o1[1])


def _precompute():
    kp = (np.uint32(0), np.uint32(42))
    kp, sub1 = _np_split(kp)
    bits1 = _np_random_bits(sub1, NUM_ITEMS)
    kp, sub2 = _np_split(kp)
    bits2 = _np_random_bits(sub2, NUM_ITEMS)
    a1 = np.argsort(bits1, kind="stable").astype(np.int32)
    a2 = np.argsort(bits2, kind="stable").astype(np.int32)
    r1g = np.empty(NUM_ITEMS, np.int32)
    r1g[a1] = np.arange(NUM_ITEMS, dtype=np.int32)
    r2g = np.empty(NUM_ITEMS, np.int32)
    r2g[a2] = np.arange(NUM_ITEMS, dtype=np.int32)
    js = np.arange(NUM_ITEMS - TOP, NUM_ITEMS, dtype=np.int32)
    o1_ = np.argsort(r1g[js], kind="stable")
    o2_ = np.argsort(r2g[js], kind="stable")
    return (a1, a2, r1g[js][o1_].copy(), js[o1_].copy(),
            r2g[js][o2_].copy(), js[o2_].copy())


_A1, _A2, _L1R, _L1J, _L2R, _L2J = _precompute()

_mesh = plsc.VectorSubcoreMesh(core_axis_name="c", subcore_axis_name="s")
_i32 = jnp.int32


def _wid():
    return lax.axis_index("s") * 2 + lax.axis_index("c")


def _iota16():
    return lax.iota(_i32, 16)


@functools.partial(
    pl.kernel,
    out_type=(
        jax.ShapeDtypeStruct((NWORDS,), _i32),
        jax.ShapeDtypeStruct((NWORDS,), _i32),
        jax.ShapeDtypeStruct((NW, 16), _i32),
    ),
    mesh=_mesh,
    compiler_params=pltpu.CompilerParams(needs_layout_passes=False),
    scratch_types=[
        pltpu.VMEM((BATCH,), _i32),
        pltpu.VMEM((VRANGE,), _i32),
        pltpu.VMEM((WPT,), _i32),
        pltpu.VMEM((WPT,), _i32),
        pltpu.VMEM((WPT,), _i32),
        pltpu.VMEM((16,), _i32),
    ],
)
def _k1(item_hbm, bits_hbm, pcl_hbm, counts_hbm,
        item_v, table_v, words_v, cnts_v, pfx_v, stage_v):
    wid = _wid()
    lo = wid * VRANGE
    iota = _iota16()
    zeros16 = jnp.zeros((16,), _i32)
    ones16 = jnp.ones((16,), _i32)

    pltpu.sync_copy(item_hbm, item_v)

    @plsc.parallel_loop(0, VRANGE // 16, unroll=8)
    def _zero_loop(i):
        table_v[pl.ds(i * 16, 16)] = zeros16

    @plsc.parallel_loop(0, BATCH // 16, unroll=4)
    def _scat_loop(i):
        v = item_v[pl.ds(i * 16, 16)]
        rel = v - lo
        msk = (rel >= 0) & (rel < VRANGE)
        relc = jnp.clip(rel, 0, VRANGE - 1)
        plsc.store_scatter(table_v, [relc], ones16, mask=msk)

    @plsc.parallel_loop(0, WPT // 16, unroll=2, carry=jnp.zeros((16,), _i32))
    def pack_body(g, du_acc):
        widx = g * 16 + iota
        base = widx * 32
        wv = zeros16
        cnt = zeros16
        for l in range(32):
            t = plsc.load_gather(table_v, [base + l])
            nz = t != 0
            bit = np.int32(np.uint32(1 << l))
            wv = wv | jnp.where(nz, jnp.full((16,), bit, _i32), zeros16)
            cnt = cnt + nz.astype(_i32)
        wglob = widx + wid * WPT
        fake = wglob >= REAL_WORDS
        wv = jnp.where(fake, jnp.full((16,), np.int32(-1), _i32), wv)
        du_acc = du_acc + jnp.where(fake, zeros16, cnt)
        cnt = jnp.where(fake, jnp.full((16,), np.int32(32), _i32), cnt)
        words_v[pl.ds(g * 16, 16)] = wv
        cnts_v[pl.ds(g * 16, 16)] = cnt
        return du_acc
    du = jnp.sum(pack_body)

    def pfx_body(g, acc):
        c = cnts_v[pl.ds(g * 16, 16)]
        s = plsc.cumsum(c)
        pfx_v[pl.ds(g * 16, 16)] = (s - c) + acc
        return acc + jnp.sum(c)
    total = lax.fori_loop(0, WPT // 16, pfx_body, jnp.int32(0))

    row = jnp.where(iota == 0, du, jnp.where(iota == 1, total, 0)).astype(_i32)
    stage_v[...] = row
    pltpu.sync_copy(stage_v, counts_hbm.at[wid])
    pltpu.sync_copy(words_v, bits_hbm.at[pl.ds(wid * WPT, WPT)])
    pltpu.sync_copy(pfx_v, pcl_hbm.at[pl.ds(wid * WPT, WPT)])


@functools.partial(
    pl.kernel,
    out_type=(
        jax.ShapeDtypeStruct((TOP,), _i32),
        jax.ShapeDtypeStruct((TOP,), _i32),
        jax.ShapeDtypeStruct((NW,), _i32),
    ),
    mesh=_mesh,
    compiler_params=pltpu.CompilerParams(needs_layout_passes=False),
    scratch_types=[
        pltpu.VMEM((NW, 16), _i32),
        pltpu.VMEM((TOP,), _i32),
        pltpu.VMEM((TOP,), _i32),
        pltpu.VMEM((TOP,), _i32),
        pltpu.VMEM((32,), _i32),
    ],
)
def _k2(counts_hbm, l1r_hbm, l1j_hbm, l2r_hbm, l2j_hbm,
        e1_hbm, e2_hbm, pcoff_hbm,
        counts_v, lr_v, lj_v, e_v, stage_v):
    wid = _wid()
    iota = _iota16()
    big16 = jnp.full((16,), np.int32(BIG), _i32)

    pltpu.sync_copy(counts_hbm, counts_v)

    du = jnp.int32(0)
    for r in range(NW):
        v = counts_v[r]
        du = du + jnp.sum(jnp.where(iota == 0, v, 0))
    n = jnp.int32(NUM_ITEMS) - du

    def build_e(r_hbm, j_hbm, out_hbm):
        pltpu.sync_copy(r_hbm, lr_v)
        pltpu.sync_copy(j_hbm, lj_v)

        def fill_body(i, _):
            e_v[pl.ds(i * 16, 16)] = big16
            return 0
        lax.fori_loop(0, TOP // 16, fill_body, 0)

        def comp_body(i, off_v):
            vj = lj_v[pl.ds(i * 16, 16)]
            vr = lr_v[pl.ds(i * 16, 16)]
            m = vj >= n
            ci = plsc.cumsum(m.astype(_i32))
            idx = off_v + ci - 1
            idxc = jnp.maximum(idx, 0)
            plsc.store_scatter(e_v, [idxc], vr - idxc, mask=m)
            return off_v + plsc.all_reduce_population_count(m)
        lax.fori_loop(0, TOP // 16, comp_body, jnp.zeros((16,), _i32))
        pltpu.sync_copy(e_v, out_hbm)

    @pl.when(wid == 0)
    def _():
        build_e(l1r_hbm, l1j_hbm, e1_hbm)

    @pl.when(wid == 1)
    def _():
        build_e(l2r_hbm, l2j_hbm, e2_hbm)

    @pl.when(wid == 2)
    def _():
        vec0 = jnp.zeros((16,), _i32)
        vec1 = jnp.zeros((16,), _i32)
        acc = jnp.int32(0)
        for r in range(NW):
            v = counts_v[r]
            t = jnp.sum(jnp.where(iota == 1, v, 0))
            if r < 16:
                vec0 = jnp.where(iota == r, acc, vec0)
            else:
                vec1 = jnp.where(iota == (r - 16), acc, vec1)
            acc = acc + t
        stage_v[pl.ds(0, 16)] = vec0
        stage_v[pl.ds(16, 16)] = vec1
        pltpu.sync_copy(stage_v, pcoff_hbm)


def _bsearch(e_ref, q):
    lo = jnp.zeros((16,), _i32)
    hi = jnp.full((16,), np.int32(TOP), _i32)
    for _ in range(15):
        mid = (lo + hi) >> 1
        v = plsc.load_gather(e_ref, [mid])
        go = v <= q
        lo = jnp.where(go, mid + 1, lo)
        hi = jnp.where(go, hi, mid)
    return lo


@functools.partial(
    pl.kernel,
    out_type=jax.ShapeDtypeStruct((NUM_NEG,), _i32),
    mesh=_mesh,
    compiler_params=pltpu.CompilerParams(needs_layout_passes=False),
    scratch_types=[
        pltpu.VMEM((TOP,), _i32),
        pltpu.VMEM((TOP,), _i32),
        pltpu.VMEM((NWORDS,), _i32),
        pltpu.VMEM((NWORDS,), _i32),
        pltpu.VMEM((NW,), _i32),
        pltpu.VMEM((QPT,), _i32),
        pltpu.VMEM((16, 128), _i32),
        pltpu.VMEM((QPT,), _i32),
        pltpu.VMEM((QPT,), _i32),
        pltpu.VMEM((QPT,), _i32),
        pltpu.SemaphoreType.DMA,
    ],
)
def _k3(e1_hbm, e2_hbm, bits_hbm, pcl_hbm, pcoff_hbm, a1_hbm, a2_hbm,
        neg_hbm,
        e1_v, e2_v, bits_v, pcg_v, pcoff_v, idxf_v, idx2_v, g2_v, g1_v,
        out_v, sem):
    wid = _wid()
    iota = _iota16()
    kbase = wid * QPT

    pltpu.sync_copy(e1_hbm, e1_v)
    pltpu.sync_copy(e2_hbm, e2_v)
    pltpu.sync_copy(bits_hbm, bits_v)
    pltpu.sync_copy(pcl_hbm, pcg_v)
    pltpu.sync_copy(pcoff_hbm, pcoff_v)

    @plsc.parallel_loop(0, NWORDS // 16, unroll=4)
    def _glob_loop(i):
        off = plsc.load_gather(pcoff_v, [jnp.full((16,), lax.div(i, 64), _i32)])
        x = pcg_v[pl.ds(i * 16, 16)]
        pcg_v[pl.ds(i * 16, 16)] = x + off

    @plsc.parallel_loop(0, QPT // 16, unroll=2)
    def _q1_loop(g):
        k = kbase + g * 16 + iota
        off2 = _bsearch(e2_v, k)
        idxf_v[pl.ds(g * 16, 16)] = k + off2

    def stage_idx():
        for r in range(16):
            for c in range(8):
                idx2_v[r, pl.ds(c * 16, 16)] = idxf_v[pl.ds(r * 128 + c * 16, 16)]

    def fire_gather(src_hbm, dst_v):
        cps = [pltpu.async_copy(src_hbm.at[idx2_v.at[r]],
                                dst_v.at[pl.ds(r * 128, 128)], sem)
               for r in range(16)]
        for cp in cps:
            cp.wait()

    stage_idx()
    fire_gather(a2_hbm, g1_v)

    @plsc.parallel_loop(0, QPT // 16, unroll=2)
    def _q2_loop(g):
        i_v = g1_v[pl.ds(g * 16, 16)]
        off1 = _bsearch(e1_v, i_v)
        idxf_v[pl.ds(g * 16, 16)] = i_v + off1

    stage_idx()
    fire_gather(a1_hbm, g2_v)

    srl = lax.shift_right_logical

    @plsc.parallel_loop(0, QPT // 16, unroll=2)
    def _q3_loop(g):
        p = g2_v[pl.ds(g * 16, 16)]
        j = p >> 5
        for s in (512, 256, 128, 64, 32, 16, 8, 4, 2, 1):
            cand = j + s
            pcv = plsc.load_gather(pcg_v, [cand])
            z = (cand << 5) - pcv
            j = jnp.where(z <= p, cand, j)
        pcj = plsc.load_gather(pcg_v, [j])
        t = p - ((j << 5) - pcj)
        w = plsc.load_gather(bits_v, [j])
        winv = ~w
        b2 = winv - (srl(winv, 1) & 0x55555555)
        b4 = (b2 & 0x33333333) + (srl(b2, 2) & 0x33333333)
        b8 = (b4 + srl(b4, 4)) & 0x0F0F0F0F
        base = jnp.zeros((16,), _i32)
        c16 = (b8 & 0xFF) + (srl(b8, 8) & 0xFF)
        go = (t >= c16).astype(_i32)
        t = t - go * c16
        base = base + go * 16
        c8 = srl(b8, base) & 0xFF
        go = (t >= c8).astype(_i32)
        t = t - go * c8
        base = base + go * 8
        c4 = srl(b4, base) & 0x0F
        go = (t >= c4).astype(_i32)
        t = t - go * c4
        base = base + go * 4
        c2 = srl(b2, base) & 0x03
        go = (t >= c2).astype(_i32)
        t = t - go * c2
        base = base + go * 2
        c1 = srl(winv, base) & 1
        go = (t >= c1).astype(_i32)
        base = base + go
        out_v[pl.ds(g * 16, 16)] = (j << 5) + base

    pltpu.sync_copy(out_v, neg_hbm.at[pl.ds(wid * QPT, QPT)])


def kernel(item, rating):
    a1 = jnp.asarray(_A1)
    a2 = jnp.asarray(_A2)
    l1r = jnp.asarray(_L1R)
    l1j = jnp.asarray(_L1J)
    l2r = jnp.asarray(_L2R)
    l2j = jnp.asarray(_L2J)

    bits, pcl, counts = _k1(item)
    e1, e2, pcoff = _k2(counts, l1r, l1j, l2r, l2j)
    neg = _k3(e1, e2, bits, pcl, pcoff, a1, a2)

    out_item = jnp.concatenate([item, neg])
    out_rating = jnp.concatenate([rating, jnp.zeros((NUM_NEG,), rating.dtype)])
    return out_item, out_rating
